# Initial kernel scaffold; baseline (speedup 1.0000x reference)
import jax
import jax.numpy as jnp
from jax.experimental import pallas as pl


def kernel(x, edge_index, cheb_W, cheb_b, W1, b1, W2, b2, W3, b3):
    raise NotImplementedError("write your pallas kernel here")


# speedup vs baseline: 18.7379x; 18.7379x over previous
import functools

import jax
import jax.numpy as jnp
from jax import lax
from jax.experimental import pallas as pl
from jax.experimental.pallas import tpu as pltpu
from jax.experimental.pallas import tpu_sc as plsc

N = 10000
E = 320000
F = 128
C = 8
K = 4
W32 = K * C
NP = 10240
RPT = NP // 16
NTILES = 32
EPT_A = E // NTILES
CHUNK = 128
CHUNKS = 79
EPT_H = CHUNKS * CHUNK
EPAD = NTILES * EPT_H

_mesh = plsc.VectorSubcoreMesh(core_axis_name="c", subcore_axis_name="s")


@functools.partial(
    pl.kernel,
    mesh=_mesh,
    out_type=jax.ShapeDtypeStruct((NTILES, NP), jnp.float32),
    compiler_params=pltpu.CompilerParams(needs_layout_passes=False),
    scratch_types=[
        pltpu.VMEM((EPT_A,), jnp.int32),
        pltpu.VMEM((NP,), jnp.float32),
    ],
)
def _deg_kernel(src_hbm, degp_hbm, idx_v, deg_v):
    c = lax.axis_index("c")
    s = lax.axis_index("s")
    w = c * 16 + s
    pltpu.sync_copy(src_hbm.at[w], idx_v)
    zeros16 = jnp.zeros((16,), jnp.float32)
    ones16 = jnp.ones((16,), jnp.float32)

    def zbody(i, carry):
        deg_v[pl.ds(i * 16, 16)] = zeros16
        return carry

    lax.fori_loop(0, NP // 16, zbody, 0)

    def ebody(e, carry):
        idx = idx_v[pl.ds(e * 16, 16)]
        plsc.addupdate_scatter(deg_v, [idx], ones16)
        return carry

    lax.fori_loop(0, EPT_A // 16, ebody, 0)
    pltpu.sync_copy(deg_v, degp_hbm.at[w])


@functools.partial(
    pl.kernel,
    mesh=_mesh,
    out_type=jax.ShapeDtypeStruct((2, NP, W32), jnp.float32),
    compiler_params=pltpu.CompilerParams(use_tc_tiling_on_sc=False),
    scratch_types=[
        pltpu.VMEM((CHUNKS, CHUNK), jnp.int32),
        pltpu.VMEM((CHUNKS, CHUNK), jnp.int32),
        pltpu.VMEM((CHUNK, W32), jnp.float32),
        pltpu.VMEM_SHARED((NP, W32), jnp.float32),
        pltpu.SemaphoreType.DMA,
    ],
)
def _hop_kernel(g_hbm, srcs_hbm, dsts_hbm, zeros_hbm, out_hbm,
                src_v, dst_v, rows_v, acc, gsem):
    c = lax.axis_index("c")
    s = lax.axis_index("s")
    w = c * 16 + s
    pltpu.sync_copy(srcs_hbm.at[w], src_v)
    pltpu.sync_copy(dsts_hbm.at[w], dst_v)
    pltpu.sync_copy(zeros_hbm.at[pl.ds(s * RPT, RPT)],
                    acc.at[pl.ds(s * RPT, RPT)])
    plsc.subcore_barrier()
    for j in range(CHUNKS):
        pltpu.async_copy(g_hbm.at[src_v.at[j]], rows_v, gsem).wait()
        pltpu.sync_copy(rows_v, acc.at[dst_v.at[j]], add=True)
    plsc.subcore_barrier()
    pltpu.sync_copy(acc.at[pl.ds(s * RPT, RPT)],
                    out_hbm.at[c].at[pl.ds(s * RPT, RPT)])


def _proj_body(degp_ref, x_ref, ws_ref, s0_ref, g0_ref, dis_ref):
    deg = jnp.sum(degp_ref[...], axis=1, keepdims=True)
    dis = jnp.where(deg > 0.0,
                    lax.rsqrt(jnp.maximum(deg, 1.0)),
                    jnp.zeros_like(deg))
    dis2 = jnp.broadcast_to(dis, (NP, W32))
    y = jnp.dot(x_ref[...], ws_ref[...], preferred_element_type=jnp.float32)
    s0_ref[0:N, :] = y
    s0_ref[N:NP, :] = jnp.zeros((NP - N, W32), jnp.float32)
    g0_ref[0:N, :] = dis2[0:N, :] * y
    g0_ref[N:NP, :] = jnp.zeros((NP - N, W32), jnp.float32)
    dis_ref[...] = dis2


_proj_call = pl.pallas_call(
    _proj_body,
    out_shape=[
        jax.ShapeDtypeStruct((NP, W32), jnp.float32),
        jax.ShapeDtypeStruct((NP, W32), jnp.float32),
        jax.ShapeDtypeStruct((NP, W32), jnp.float32),
    ],
)


def _upd1_body(aggp_ref, dis_ref, s1_ref, g1_ref):
    agg = aggp_ref[0] + aggp_ref[1]
    dis2 = dis_ref[...]
    s1 = -dis2 * agg
    s1_ref[...] = s1
    g1_ref[...] = dis2 * s1


_upd1_call = pl.pallas_call(
    _upd1_body,
    out_shape=[
        jax.ShapeDtypeStruct((NP, W32), jnp.float32),
        jax.ShapeDtypeStruct((NP, W32), jnp.float32),
    ],
)


def _upd2_body(aggp_ref, dis_ref, sprev_ref, s2_ref, g2_ref):
    agg = aggp_ref[0] + aggp_ref[1]
    dis2 = dis_ref[...]
    s2 = -2.0 * dis2 * agg - sprev_ref[...]
    s2_ref[...] = s2
    g2_ref[...] = dis2 * s2


_upd2_call = pl.pallas_call(
    _upd2_body,
    out_shape=[
        jax.ShapeDtypeStruct((NP, W32), jnp.float32),
        jax.ShapeDtypeStruct((NP, W32), jnp.float32),
    ],
)


def _head_body(aggp_ref, dis_ref, s0_ref, s1_ref, s2_ref, cb_ref,
               w1_ref, b1_ref, w2_ref, b2_ref, w3_ref, b3_ref, out_ref):
    agg = aggp_ref[0] + aggp_ref[1]
    s3 = -2.0 * dis_ref[...] * agg - s1_ref[...]
    feat = (s0_ref[0:N, 0:C] + s1_ref[0:N, C:2 * C]
            + s2_ref[0:N, 2 * C:3 * C] + s3[0:N, 3 * C:4 * C]
            + cb_ref[...])
    feat = jnp.maximum(feat, 0.0)
    pooled = jnp.sum(feat, axis=0, keepdims=True)
    h = jnp.dot(pooled, w1_ref[...], preferred_element_type=jnp.float32)
    h = jnp.maximum(h + b1_ref[...], 0.0)
    h = jnp.dot(h, w2_ref[...], preferred_element_type=jnp.float32)
    h = jnp.maximum(h + b2_ref[...], 0.0)
    h = jnp.dot(h, w3_ref[...], preferred_element_type=jnp.float32)
    out_ref[...] = h + b3_ref[...]


_head_call = pl.pallas_call(
    _head_body,
    out_shape=jax.ShapeDtypeStruct((1, 1), jnp.float32),
)


def kernel(x, edge_index, cheb_W, cheb_b, W1, b1, W2, b2, W3, b3):
    src = edge_index[0].astype(jnp.int32)
    dst = edge_index[1].astype(jnp.int32)
    pad = jnp.full((EPAD - E,), N, dtype=jnp.int32)
    src3 = jnp.concatenate([src, pad]).reshape(NTILES, CHUNKS, CHUNK)
    dst3 = jnp.concatenate([dst, pad]).reshape(NTILES, CHUNKS, CHUNK)
    src2 = src.reshape(NTILES, EPT_A)
    zeros_nw = jnp.zeros((NP, W32), jnp.float32)
    ws = jnp.transpose(cheb_W, (1, 0, 2)).reshape(F, K * C)

    degp = _deg_kernel(src2)
    degp_t = jnp.transpose(degp)
    s0, g0, dis2 = _proj_call(degp_t, x, ws)
    agg1 = _hop_kernel(g0, src3, dst3, zeros_nw)
    s1, g1 = _upd1_call(agg1, dis2)
    agg2 = _hop_kernel(g1, src3, dst3, zeros_nw)
    s2, g2 = _upd2_call(agg2, dis2, s0)
    agg3 = _hop_kernel(g2, src3, dst3, zeros_nw)
    return _head_call(agg3, dis2, s0, s1, s2,
                      cheb_b.reshape(1, C), W1, b1.reshape(1, 32),
                      W2, b2.reshape(1, 16), W3, b3.reshape(1, 1))


# speedup vs baseline: 31.3007x; 1.6704x over previous
import functools

import jax
import jax.numpy as jnp
from jax import lax
from jax.experimental import pallas as pl
from jax.experimental.pallas import tpu as pltpu
from jax.experimental.pallas import tpu_sc as plsc

N = 10000
E = 320000
F = 128
C = 8
K = 4
W32 = K * C
NP = 10240
RPT = NP // 16
NTILES = 32
EPT_A = E // NTILES
CHUNK = 128
CHUNKS = 79
EPT_H = CHUNKS * CHUNK
EPAD = NTILES * EPT_H
NBUF = 4

_mesh = plsc.VectorSubcoreMesh(core_axis_name="c", subcore_axis_name="s")


@functools.partial(
    pl.kernel,
    mesh=_mesh,
    out_type=jax.ShapeDtypeStruct((NTILES, NP), jnp.float32),
    compiler_params=pltpu.CompilerParams(needs_layout_passes=False),
    scratch_types=[
        pltpu.VMEM((EPT_A,), jnp.int32),
        pltpu.VMEM((NP,), jnp.float32),
    ],
)
def _deg_kernel(src_hbm, degp_hbm, idx_v, deg_v):
    c = lax.axis_index("c")
    s = lax.axis_index("s")
    w = c * 16 + s
    pltpu.sync_copy(src_hbm.at[w], idx_v)
    zeros16 = jnp.zeros((16,), jnp.float32)
    ones16 = jnp.ones((16,), jnp.float32)

    def zbody(i, carry):
        deg_v[pl.ds(i * 16, 16)] = zeros16
        return carry

    lax.fori_loop(0, NP // 16, zbody, 0)

    def ebody(e, carry):
        idx = idx_v[pl.ds(e * 16, 16)]
        plsc.addupdate_scatter(deg_v, [idx], ones16)
        return carry

    lax.fori_loop(0, EPT_A // 16, ebody, 0)
    pltpu.sync_copy(deg_v, degp_hbm.at[w])


def _make_hop(width):
    @functools.partial(
        pl.kernel,
        mesh=_mesh,
        out_type=jax.ShapeDtypeStruct((2, NP, width), jnp.float32),
        compiler_params=pltpu.CompilerParams(use_tc_tiling_on_sc=False),
        scratch_types=[
            pltpu.VMEM((CHUNKS, CHUNK), jnp.int32),
            pltpu.VMEM((CHUNKS, CHUNK), jnp.int32),
            pltpu.VMEM((NBUF, CHUNK, width), jnp.float32),
            pltpu.VMEM_SHARED((NP, width), jnp.float32),
        ] + [pltpu.SemaphoreType.DMA] * (2 * NBUF),
    )
    def _hop_kernel(g_hbm, srcs_hbm, dsts_hbm, zeros_hbm, out_hbm,
                    src_v, dst_v, rows_v, acc, *sems):
        gsem = sems[:NBUF]
        ssem = sems[NBUF:]
        c = lax.axis_index("c")
        s = lax.axis_index("s")
        w = c * 16 + s
        pltpu.sync_copy(srcs_hbm.at[w], src_v)
        pltpu.sync_copy(dsts_hbm.at[w], dst_v)
        pltpu.sync_copy(zeros_hbm.at[pl.ds(s * RPT, RPT)],
                        acc.at[pl.ds(s * RPT, RPT)])
        plsc.subcore_barrier()
        gh = [None] * NBUF
        sh = [None] * NBUF
        for b in range(NBUF - 1):
            gh[b] = pltpu.async_copy(g_hbm.at[src_v.at[b]], rows_v.at[b],
                                     gsem[b])
        for j in range(CHUNKS):
            b = j % NBUF
            g = j + NBUF - 1
            if g < CHUNKS:
                gb = g % NBUF
                if sh[gb] is not None:
                    sh[gb].wait()
                gh[gb] = pltpu.async_copy(g_hbm.at[src_v.at[g]],
                                          rows_v.at[gb], gsem[gb])
            gh[b].wait()
            sh[b] = pltpu.async_copy(rows_v.at[b], acc.at[dst_v.at[j]],
                                     ssem[b], add=True)
        for b in range(NBUF):
            if sh[b] is not None:
                sh[b].wait()
        plsc.subcore_barrier()
        pltpu.sync_copy(acc.at[pl.ds(s * RPT, RPT)],
                        out_hbm.at[c].at[pl.ds(s * RPT, RPT)])

    return _hop_kernel


_hop24 = _make_hop(24)
_hop16 = _make_hop(16)
_hop8 = _make_hop(8)


def _proj_body(degp_ref, x_ref, ws_ref, s0_ref, g0_ref, dis_ref):
    deg = jnp.sum(degp_ref[...], axis=1, keepdims=True)
    dis = jnp.where(deg > 0.0,
                    lax.rsqrt(jnp.maximum(deg, 1.0)),
                    jnp.zeros_like(deg))
    dis24 = jnp.broadcast_to(dis, (NP, 24))
    y = jnp.dot(x_ref[...], ws_ref[...], preferred_element_type=jnp.float32)
    s0_ref[0:N, :] = y
    s0_ref[N:NP, :] = jnp.zeros((NP - N, W32), jnp.float32)
    g0_ref[0:N, :] = dis24[0:N, :] * y[:, C:W32]
    g0_ref[N:NP, :] = jnp.zeros((NP - N, 24), jnp.float32)
    dis_ref[...] = dis24


_proj_call = pl.pallas_call(
    _proj_body,
    out_shape=[
        jax.ShapeDtypeStruct((NP, W32), jnp.float32),
        jax.ShapeDtypeStruct((NP, 24), jnp.float32),
        jax.ShapeDtypeStruct((NP, 24), jnp.float32),
    ],
)


def _upd1_body(aggp_ref, dis_ref, h1_ref, g1_ref):
    agg = aggp_ref[0] + aggp_ref[1]
    dis24 = dis_ref[...]
    h1 = -dis24 * agg
    h1_ref[...] = h1
    g1_ref[...] = dis24[:, 0:16] * h1[:, C:3 * C]


_upd1_call = pl.pallas_call(
    _upd1_body,
    out_shape=[
        jax.ShapeDtypeStruct((NP, 24), jnp.float32),
        jax.ShapeDtypeStruct((NP, 16), jnp.float32),
    ],
)


def _upd2_body(aggp_ref, dis_ref, s0_ref, t2w2_ref, g2_ref):
    agg = aggp_ref[0] + aggp_ref[1]
    u2 = -dis_ref[:, 0:16] * agg
    t2w2_ref[...] = 2.0 * u2[:, 0:C] - s0_ref[:, 2 * C:3 * C]
    t2w3 = 2.0 * u2[:, C:2 * C] - s0_ref[:, 3 * C:4 * C]
    g2_ref[...] = dis_ref[:, 0:C] * t2w3


_upd2_call = pl.pallas_call(
    _upd2_body,
    out_shape=[
        jax.ShapeDtypeStruct((NP, C), jnp.float32),
        jax.ShapeDtypeStruct((NP, C), jnp.float32),
    ],
)


def _head_body(aggp_ref, dis_ref, s0_ref, h1_ref, t2w2_ref, cb_ref,
               w1_ref, b1_ref, w2_ref, b2_ref, w3_ref, b3_ref, out_ref):
    agg = aggp_ref[0] + aggp_ref[1]
    u3 = -dis_ref[:, 0:C] * agg
    t3w3 = 2.0 * u3 - h1_ref[:, 2 * C:3 * C]
    feat = (s0_ref[0:N, 0:C] + h1_ref[0:N, 0:C]
            + t2w2_ref[0:N, :] + t3w3[0:N, :]
            + cb_ref[...])
    feat = jnp.maximum(feat, 0.0)
    pooled = jnp.sum(feat, axis=0, keepdims=True)
    h = jnp.dot(pooled, w1_ref[...], preferred_element_type=jnp.float32)
    h = jnp.maximum(h + b1_ref[...], 0.0)
    h = jnp.dot(h, w2_ref[...], preferred_element_type=jnp.float32)
    h = jnp.maximum(h + b2_ref[...], 0.0)
    h = jnp.dot(h, w3_ref[...], preferred_element_type=jnp.float32)
    out_ref[...] = h + b3_ref[...]


_head_call = pl.pallas_call(
    _head_body,
    out_shape=jax.ShapeDtypeStruct((1, 1), jnp.float32),
)


def kernel(x, edge_index, cheb_W, cheb_b, W1, b1, W2, b2, W3, b3):
    src = edge_index[0].astype(jnp.int32)
    dst = edge_index[1].astype(jnp.int32)
    pad = jnp.full((EPAD - E,), N, dtype=jnp.int32)
    src3 = jnp.concatenate([src, pad]).reshape(NTILES, CHUNKS, CHUNK)
    dst3 = jnp.concatenate([dst, pad]).reshape(NTILES, CHUNKS, CHUNK)
    src2 = src.reshape(NTILES, EPT_A)
    zeros24 = jnp.zeros((NP, 24), jnp.float32)
    zeros16 = jnp.zeros((NP, 16), jnp.float32)
    zeros8 = jnp.zeros((NP, C), jnp.float32)
    ws = jnp.transpose(cheb_W, (1, 0, 2)).reshape(F, K * C)

    degp = _deg_kernel(src2)
    degp_t = jnp.transpose(degp)
    s0, g0, dis24 = _proj_call(degp_t, x, ws)
    agg1 = _hop24(g0, src3, dst3, zeros24)
    h1, g1 = _upd1_call(agg1, dis24)
    agg2 = _hop16(g1, src3, dst3, zeros16)
    t2w2, g2 = _upd2_call(agg2, dis24, s0)
    agg3 = _hop8(g2, src3, dst3, zeros8)
    return _head_call(agg3, dis24, s0, h1, t2w2,
                      cheb_b.reshape(1, C), W1, b1.reshape(1, 32),
                      W2, b2.reshape(1, 16), W3, b3.reshape(1, 1))


# speedup vs baseline: 36.1932x; 1.1563x over previous
import functools

import jax
import jax.numpy as jnp
from jax import lax
from jax.experimental import pallas as pl
from jax.experimental.pallas import tpu as pltpu
from jax.experimental.pallas import tpu_sc as plsc

N = 10000
E = 320000
F = 128
C = 8
K = 4
W32 = K * C
NP = 10240
RPT = NP // 16
NTILES = 32
EPT_A = E // NTILES
CHUNK = 128
CH0A = 132
CH1A = 26
CH0B = 108
CH1B = 50
EPAD = 16 * (CH0A + CH1A) * CHUNK
NBUF = 8
LA = 4
FR = NP * W32 // 128
XR = N // 4

_mesh = plsc.VectorSubcoreMesh(core_axis_name="c", subcore_axis_name="s")


@functools.partial(
    pl.kernel,
    mesh=_mesh,
    out_type=jax.ShapeDtypeStruct((NTILES, NP), jnp.float32),
    compiler_params=pltpu.CompilerParams(needs_layout_passes=False),
    scratch_types=[
        pltpu.VMEM((EPT_A,), jnp.int32),
        pltpu.VMEM((NP,), jnp.float32),
    ],
)
def _deg_kernel(src_hbm, degp_hbm, idx_v, deg_v):
    c = lax.axis_index("c")
    s = lax.axis_index("s")
    w = c * 16 + s
    pltpu.sync_copy(src_hbm.at[w], idx_v)
    zeros16 = jnp.zeros((16,), jnp.float32)
    ones16 = jnp.ones((16,), jnp.float32)

    def zbody(i, carry):
        deg_v[pl.ds(i * 16, 16)] = zeros16
        return carry

    lax.fori_loop(0, NP // 16, zbody, 0)

    def ebody(e, carry):
        idx = idx_v[pl.ds(e * 16, 16)]
        plsc.addupdate_scatter(deg_v, [idx], ones16)
        return carry

    lax.fori_loop(0, EPT_A // 16, ebody, 0)
    pltpu.sync_copy(deg_v, degp_hbm.at[w])


def _make_hop(width, ch0, ch1):
    @functools.partial(
        pl.kernel,
        mesh=_mesh,
        out_type=jax.ShapeDtypeStruct((2, NP, width), jnp.float32),
        compiler_params=pltpu.CompilerParams(use_tc_tiling_on_sc=False),
        scratch_types=[
            pltpu.VMEM((ch0, CHUNK), jnp.int32),
            pltpu.VMEM((ch0, CHUNK), jnp.int32),
            pltpu.VMEM((NBUF, CHUNK, width), jnp.float32),
            pltpu.VMEM_SHARED((NP, width), jnp.float32),
        ] + [pltpu.SemaphoreType.DMA] * (2 * NBUF),
    )
    def _hop_kernel(g_hbm, srcs0_hbm, dsts0_hbm, srcs1_hbm, dsts1_hbm,
                    zeros_hbm, out_hbm, src_v, dst_v, rows_v, acc, *sems):
        gsem = sems[:NBUF]
        ssem = sems[NBUF:]
        c = lax.axis_index("c")
        s = lax.axis_index("s")
        pltpu.sync_copy(zeros_hbm.at[pl.ds(s * RPT, RPT)],
                        acc.at[pl.ds(s * RPT, RPT)])

        def run(nchunks, nbuf, la):
            gh = [None] * nbuf
            sh = [None] * nbuf
            for g in range(min(la, nchunks)):
                gh[g % nbuf] = pltpu.async_copy(
                    g_hbm.at[src_v.at[g]], rows_v.at[g % nbuf],
                    gsem[g % nbuf])
            for j in range(nchunks):
                b = j % nbuf
                g = j + la
                if g < nchunks:
                    gb = g % nbuf
                    if sh[gb] is not None:
                        sh[gb].wait()
                        sh[gb] = None
                    gh[gb] = pltpu.async_copy(g_hbm.at[src_v.at[g]],
                                              rows_v.at[gb], gsem[gb])
                gh[b].wait()
                sh[b] = pltpu.async_copy(rows_v.at[b],
                                         acc.at[dst_v.at[j]],
                                         ssem[b], add=True)
            for b in range(nbuf):
                if sh[b] is not None:
                    sh[b].wait()

        @pl.when(c == 0)
        def _():
            pltpu.sync_copy(srcs0_hbm.at[s], src_v.at[pl.ds(0, ch0)])
            pltpu.sync_copy(dsts0_hbm.at[s], dst_v.at[pl.ds(0, ch0)])
            run(ch0, NBUF, LA)

        @pl.when(c == 1)
        def _():
            pltpu.sync_copy(srcs1_hbm.at[s], src_v.at[pl.ds(0, ch1)])
            pltpu.sync_copy(dsts1_hbm.at[s], dst_v.at[pl.ds(0, ch1)])
            run(ch1, 4, 3)

        plsc.subcore_barrier()
        pltpu.sync_copy(acc.at[pl.ds(s * RPT, RPT)],
                        out_hbm.at[c].at[pl.ds(s * RPT, RPT)])

    return _hop_kernel


_hop32 = _make_hop(32, CH0A, CH1A)
_hop16 = _make_hop(16, CH0B, CH1B)
_hop8 = _make_hop(8, CH0B, CH1B)


def _dis_body(degp_ref, dis_ref):
    deg = jnp.sum(degp_ref[...], axis=0, keepdims=True)
    dis = jnp.where(deg > 0.0,
                    lax.rsqrt(jnp.maximum(deg, 1.0)),
                    jnp.zeros_like(deg))
    dis_ref[...] = jnp.broadcast_to(dis, (8, NP))


_dis_call = pl.pallas_call(
    _dis_body,
    out_shape=jax.ShapeDtypeStruct((8, NP), jnp.float32),
)


def _proj_body(x4_ref, w4_ref, s0_ref):
    y = jnp.dot(x4_ref[...], w4_ref[...], preferred_element_type=jnp.float32)
    s0_ref[0:XR, :] = y
    s0_ref[XR:FR, :] = jnp.zeros((FR - XR, 128), jnp.float32)


_proj_call = pl.pallas_call(
    _proj_body,
    out_shape=jax.ShapeDtypeStruct((FR, 128), jnp.float32),
)


def _g0_body(disf_ref, s0_ref, g0_ref):
    g0_ref[...] = disf_ref[...] * s0_ref[...]


_g0_call = pl.pallas_call(
    _g0_body,
    out_shape=jax.ShapeDtypeStruct((FR, 128), jnp.float32),
)


def _upd1_body(aggf_ref, disf_ref, h1_ref, g1_ref):
    disf = disf_ref[...]
    h1 = -disf * (aggf_ref[0] + aggf_ref[1])
    h1_ref[...] = h1
    g1_ref[...] = disf * h1


_upd1_call = pl.pallas_call(
    _upd1_body,
    out_shape=[
        jax.ShapeDtypeStruct((FR, 128), jnp.float32),
        jax.ShapeDtypeStruct((FR, 128), jnp.float32),
    ],
)


def _upd2_body(aggf_ref, disf16_ref, s016_ref, t2_ref, g2_ref):
    disf = disf16_ref[...]
    t2 = -2.0 * disf * (aggf_ref[0] + aggf_ref[1]) - s016_ref[...]
    t2_ref[...] = t2
    g2_ref[...] = disf * t2


_upd2_call = pl.pallas_call(
    _upd2_body,
    out_shape=[
        jax.ShapeDtypeStruct((NP * 16 // 128, 128), jnp.float32),
        jax.ShapeDtypeStruct((NP * 16 // 128, 128), jnp.float32),
    ],
)


def _head_body(aggf_ref, disf8_ref, h1w3_ref, s0w0_ref, h1w1_ref,
               t2w2_ref, cb128_ref,
               w1_ref, b1_ref, w2_ref, b2_ref, w3_ref, b3_ref, out_ref):
    t3 = (-2.0 * disf8_ref[...] * (aggf_ref[0] + aggf_ref[1])
          - h1w3_ref[...])
    feat = (s0w0_ref[...] + h1w1_ref[...] + t2w2_ref[...] + t3
            + cb128_ref[...])
    feat = jnp.maximum(feat, 0.0)
    red = jnp.sum(feat[0:N * 8 // 128, :], axis=0, keepdims=True)
    pooled = jnp.zeros((1, C), jnp.float32)
    for k in range(16):
        pooled = pooled + red[:, C * k:C * (k + 1)]
    h = jnp.dot(pooled, w1_ref[...], preferred_element_type=jnp.float32)
    h = jnp.maximum(h + b1_ref[...], 0.0)
    h = jnp.dot(h, w2_ref[...], preferred_element_type=jnp.float32)
    h = jnp.maximum(h + b2_ref[...], 0.0)
    h = jnp.dot(h, w3_ref[...], preferred_element_type=jnp.float32)
    out_ref[...] = h + b3_ref[...]


_head_call = pl.pallas_call(
    _head_body,
    out_shape=jax.ShapeDtypeStruct((1, 1), jnp.float32),
)


def kernel(x, edge_index, cheb_W, cheb_b, W1, b1, W2, b2, W3, b3):
    src = edge_index[0].astype(jnp.int32)
    dst = edge_index[1].astype(jnp.int32)
    pad = jnp.full((EPAD - E,), N, dtype=jnp.int32)
    srcp = jnp.concatenate([src, pad])
    dstp = jnp.concatenate([dst, pad])
    e0a = 16 * CH0A * CHUNK
    srcs0a = srcp[:e0a].reshape(16, CH0A, CHUNK)
    dsts0a = dstp[:e0a].reshape(16, CH0A, CHUNK)
    srcs1a = srcp[e0a:].reshape(16, CH1A, CHUNK)
    dsts1a = dstp[e0a:].reshape(16, CH1A, CHUNK)
    e0b = 16 * CH0B * CHUNK
    srcs0b = srcp[:e0b].reshape(16, CH0B, CHUNK)
    dsts0b = dstp[:e0b].reshape(16, CH0B, CHUNK)
    srcs1b = srcp[e0b:].reshape(16, CH1B, CHUNK)
    dsts1b = dstp[e0b:].reshape(16, CH1B, CHUNK)
    src2 = src.reshape(NTILES, EPT_A)
    zeros32 = jnp.zeros((NP, 32), jnp.float32)
    zeros16 = jnp.zeros((NP, 16), jnp.float32)
    zeros8 = jnp.zeros((NP, 8), jnp.float32)
    ws = jnp.transpose(cheb_W, (1, 0, 2)).reshape(F, K * C)
    w4 = jnp.kron(jnp.eye(4, dtype=jnp.float32), ws)
    x4 = x.reshape(XR, 4 * F)
    cb128 = jnp.tile(cheb_b, 16).reshape(1, 128)

    degp = _deg_kernel(src2)
    dis_row = _dis_call(degp)[0]
    disf32 = jnp.broadcast_to(
        dis_row.reshape(FR, 4, 1), (FR, 4, 32)).reshape(FR, 128)
    disf16 = jnp.broadcast_to(
        dis_row.reshape(NP * 16 // 128, 8, 1),
        (NP * 16 // 128, 8, 16)).reshape(NP * 16 // 128, 128)
    disf8 = jnp.broadcast_to(
        dis_row.reshape(NP * 8 // 128, 16, 1),
        (NP * 8 // 128, 16, 8)).reshape(NP * 8 // 128, 128)
    s0 = _proj_call(x4, w4)
    g0 = _g0_call(disf32, s0)

    agg1 = _hop32(g0.reshape(NP, 32), srcs0a, dsts0a, srcs1a, dsts1a,
                  zeros32).reshape(2, FR, 128)
    h1, g1w = _upd1_call(agg1, disf32)
    g1 = jnp.asarray(g1w.reshape(NP, 32)[:, 16:32])
    s016 = s0.reshape(NP, 32)[:, 16:32].reshape(NP * 16 // 128, 128)
    h1w1 = h1.reshape(NP, 32)[:, 8:16].reshape(NP * 8 // 128, 128)
    h1w3 = h1.reshape(NP, 32)[:, 24:32].reshape(NP * 8 // 128, 128)
    s0w0 = s0.reshape(NP, 32)[:, 0:8].reshape(NP * 8 // 128, 128)

    agg2 = _hop16(g1, srcs0b, dsts0b, srcs1b, dsts1b,
                  zeros16).reshape(2, NP * 16 // 128, 128)
    t2, g2w = _upd2_call(agg2, disf16, s016)
    g2 = jnp.asarray(g2w.reshape(NP, 16)[:, 8:16])
    t2w2 = t2.reshape(NP, 16)[:, 0:8].reshape(NP * 8 // 128, 128)

    agg3 = _hop8(g2, srcs0b, dsts0b, srcs1b, dsts1b,
                 zeros8).reshape(2, NP * 8 // 128, 128)
    return _head_call(agg3, disf8, h1w3, s0w0, h1w1, t2w2, cb128,
                      W1, b1.reshape(1, 32),
                      W2, b2.reshape(1, 16), W3, b3.reshape(1, 1))
